# baseline (speedup 1.0000x reference)
import functools

import jax
import jax.numpy as jnp
from jax import lax
from jax.experimental import pallas as pl
from jax.experimental.pallas import tpu as pltpu
from jax.experimental.pallas import tpu_sc as plsc

NUM_EMBEDDINGS = 100000
EMBEDDING_DIM = 64
BATCH = 16384

NUM_CORES = 2
NUM_SUBCORES = 16
NUM_WORKERS = NUM_CORES * NUM_SUBCORES
B_PER_WORKER = BATCH // NUM_WORKERS


@functools.partial(
    pl.kernel,
    mesh=plsc.VectorSubcoreMesh(core_axis_name="c", subcore_axis_name="s"),
    out_type=jax.ShapeDtypeStruct((BATCH, EMBEDDING_DIM), jnp.float32),
    scratch_types=[
        pltpu.VMEM((B_PER_WORKER,), jnp.int32),
        pltpu.VMEM((B_PER_WORKER, EMBEDDING_DIM), jnp.float32),
        pltpu.SemaphoreType.DMA,
    ],
    compiler_params=pltpu.CompilerParams(
        use_tc_tiling_on_sc=False,
        skip_device_barrier=True,
        disable_bounds_checks=True,
        disable_semaphore_checks=True,
    ),
)
def _sc_gather(table_hbm, idx_hbm, out_hbm, idx_v, rows_v, sem):
    wid = lax.axis_index("s") * NUM_CORES + lax.axis_index("c")
    base = wid * B_PER_WORKER
    pltpu.sync_copy(idx_hbm.at[pl.ds(base, B_PER_WORKER)], idx_v)
    pltpu.async_copy(table_hbm.at[idx_v], rows_v, sem).wait()
    pltpu.sync_copy(rows_v, out_hbm.at[pl.ds(base, B_PER_WORKER)])


@jax.jit
def kernel(input, weight):
    return _sc_gather(weight, input.astype(jnp.int32))


# speedup vs baseline: 1.8179x; 1.8179x over previous
import functools

import jax
import jax.numpy as jnp
from jax import lax
from jax.experimental import pallas as pl
from jax.experimental.pallas import tpu as pltpu
from jax.experimental.pallas import tpu_sc as plsc

NUM_EMBEDDINGS = 100000
EMBEDDING_DIM = 64
BATCH = 16384

NUM_CORES = 2
NUM_SUBCORES = 16
NUM_WORKERS = NUM_CORES * NUM_SUBCORES
COLS_PER_WORKER = EMBEDDING_DIM // NUM_WORKERS
HALF = BATCH // 2
L = 16


@functools.partial(
    pl.kernel,
    mesh=plsc.VectorSubcoreMesh(core_axis_name="c", subcore_axis_name="s"),
    out_type=jax.ShapeDtypeStruct((EMBEDDING_DIM, BATCH), jnp.float32),
    scratch_types=[
        pltpu.VMEM((NUM_EMBEDDINGS,), jnp.float32),
        pltpu.VMEM((HALF,), jnp.int32),
        pltpu.VMEM((HALF,), jnp.float32),
    ],
    compiler_params=pltpu.CompilerParams(needs_layout_passes=False),
)
def _sc_gather(tableT, idx_hbm, outT, col_v, idx_v, out_v):
    wid = lax.axis_index("s") * NUM_CORES + lax.axis_index("c")

    def col_body(ci, carry):
        c = wid * COLS_PER_WORKER + ci
        pltpu.sync_copy(tableT.at[c], col_v)

        def half_body(h, carry2):
            pltpu.sync_copy(idx_hbm.at[pl.ds(h * HALF, HALF)], idx_v)

            def group_body(g, carry3):
                i16 = idx_v[pl.ds(g * L, L)]
                out_v[pl.ds(g * L, L)] = plsc.load_gather(col_v, [i16])
                return carry3

            lax.fori_loop(0, HALF // L, group_body, 0, unroll=4)
            pltpu.sync_copy(out_v, outT.at[c, pl.ds(h * HALF, HALF)])
            return carry2

        lax.fori_loop(0, 2, half_body, 0)
        return carry

    lax.fori_loop(0, COLS_PER_WORKER, col_body, 0)


@jax.jit
def kernel(input, weight):
    outT = _sc_gather(weight.T, input.astype(jnp.int32))
    return outT.T


# speedup vs baseline: 2.7673x; 1.5222x over previous
import functools

import jax
import jax.numpy as jnp
from jax import lax
from jax.experimental import pallas as pl
from jax.experimental.pallas import tpu as pltpu
from jax.experimental.pallas import tpu_sc as plsc

NUM_EMBEDDINGS = 100000
EMBEDDING_DIM = 64
BATCH = 16384

NUM_CORES = 2
NUM_SUBCORES = 16
NUM_WORKERS = NUM_CORES * NUM_SUBCORES
COLS_PER_WORKER = EMBEDDING_DIM // NUM_WORKERS
HALF = BATCH // 2
L = 16


@functools.partial(
    pl.kernel,
    mesh=plsc.VectorSubcoreMesh(core_axis_name="c", subcore_axis_name="s"),
    out_type=jax.ShapeDtypeStruct((EMBEDDING_DIM, BATCH), jnp.float32),
    scratch_types=[
        pltpu.VMEM((NUM_EMBEDDINGS,), jnp.float32),
        pltpu.VMEM((BATCH,), jnp.int32),
        pltpu.VMEM((HALF,), jnp.float32),
        pltpu.SemaphoreType.DMA,
    ],
    compiler_params=pltpu.CompilerParams(needs_layout_passes=False),
)
def _sc_gather(tableT, idx_hbm, outT, col_v, idx_v, out_v, sem):
    wid = lax.axis_index("s") * NUM_CORES + lax.axis_index("c")

    col0 = wid * COLS_PER_WORKER
    col_copy = pltpu.async_copy(tableT.at[col0], col_v, sem)
    pltpu.sync_copy(idx_hbm.at[pl.ds(0, BATCH)], idx_v)
    col_copy.wait()

    for ci in range(COLS_PER_WORKER):
        c = col0 + ci
        for h in range(2):

            @plsc.parallel_loop(0, HALF // L, unroll=8)
            def group_body(g):
                i16 = idx_v[pl.ds(h * HALF + g * L, L)]
                out_v[pl.ds(g * L, L)] = plsc.load_gather(col_v, [i16])

            pltpu.sync_copy(out_v, outT.at[c, pl.ds(h * HALF, HALF)])
        if ci + 1 < COLS_PER_WORKER:
            pltpu.sync_copy(tableT.at[c + 1], col_v)


@jax.jit
def kernel(input, weight):
    outT = _sc_gather(weight.T, input.astype(jnp.int32))
    return outT.T
